# baseline (speedup 1.0000x reference)
import functools

import jax
import jax.numpy as jnp
from jax import lax
from jax.experimental import pallas as pl
from jax.experimental.pallas import tpu as pltpu
from jax.experimental.pallas import tpu_sc as plsc

_NC = 2
_NS = 16
_NW = _NC * _NS
_CH = 128


@functools.lru_cache(maxsize=None)
def _make_seg_sum(n_pad: int, n_cols: int, edges_per_tile: int):
    rows_per_tile = n_pad // _NS
    n_pairs = edges_per_tile // (2 * _CH)

    mesh = plsc.VectorSubcoreMesh(core_axis_name="c", subcore_axis_name="s")

    @functools.partial(
        pl.kernel,
        out_type=jax.ShapeDtypeStruct((2, n_pad, n_cols), jnp.float32),
        mesh=mesh,
        scratch_types=[
            [pltpu.VMEM((2 * _CH,), jnp.int32) for _ in range(2)],
            pltpu.VMEM((_CH, n_cols), jnp.float32),
            pltpu.VMEM_SHARED((n_pad, n_cols), jnp.float32),
            [pltpu.SemaphoreType.DMA for _ in range(2)],
        ],
    )
    def seg(table_hbm, idx_hbm, zeros_hbm, out_hbm,
            idx, rows, acc_sp, sems):
        c = lax.axis_index("c")
        s = lax.axis_index("s")
        wid = s * _NC + c
        sem_g, sem_i = sems
        ia, ib = idx

        t0 = s * rows_per_tile
        pltpu.sync_copy(zeros_hbm.at[pl.ds(t0, rows_per_tile)],
                        acc_sp.at[pl.ds(t0, rows_per_tile)])

        base2 = wid * 2 * edges_per_tile

        def fetch(j, buf, sem):
            off = pl.multiple_of(base2 + j * 2 * _CH, 2 * _CH)
            return pltpu.async_copy(idx_hbm.at[pl.ds(off, 2 * _CH)], buf, sem)

        def work(buf):
            pltpu.async_copy(table_hbm.at[buf.at[pl.ds(0, _CH)]],
                             rows, sem_g).wait()
            pltpu.sync_copy(rows, acc_sp.at[buf.at[pl.ds(_CH, _CH)]],
                            add=True)

        plsc.subcore_barrier()

        def chunk(j, carry):
            fetch(j, ia, sem_i).wait()
            work(ia)
            return carry

        lax.fori_loop(0, 2 * n_pairs, chunk, 0)
        plsc.subcore_barrier()

        pltpu.sync_copy(acc_sp.at[pl.ds(t0, rows_per_tile)],
                        out_hbm.at[c, pl.ds(t0, rows_per_tile)])

    return seg


def _prep_edges(edge_index, junk_row):
    e = edge_index.shape[1]
    edges_per_tile = -(-e // (_NW * 2 * _CH)) * (2 * _CH)
    e_pad = _NW * edges_per_tile
    src = jnp.concatenate(
        [edge_index[0], jnp.zeros((e_pad - e,), jnp.int32)])
    dst = jnp.concatenate(
        [edge_index[1], jnp.full((e_pad - e,), junk_row, jnp.int32)])
    inter = jnp.stack(
        [src.reshape(-1, _CH), dst.reshape(-1, _CH)], axis=1).reshape(-1)
    inter = jnp.concatenate([inter, jnp.zeros((4 * _CH,), jnp.int32)])
    return inter, edges_per_tile


def _mm_body(x_ref, w_ref, o_ref):
    o_ref[...] = jnp.dot(x_ref[...], w_ref[...],
                         preferred_element_type=jnp.float32)


def _matmul(x, w, block_rows, n_out):
    n, d = x.shape
    k = w.shape[1]
    return pl.pallas_call(
        _mm_body,
        grid=(n // block_rows,),
        in_specs=[pl.BlockSpec((block_rows, d), lambda i: (i, 0)),
                  pl.BlockSpec((d, k), lambda i: (0, 0))],
        out_specs=pl.BlockSpec((block_rows, k), lambda i: (i, 0)),
        out_shape=jax.ShapeDtypeStruct((n_out, k), jnp.float32),
    )(x, w)


def _enc2_body(pa_ref, pb_ref, wa_ref, wb_ref, o_ref):
    ha = jnp.maximum(pa_ref[0][:, :64] + pa_ref[1][:, :64], 0.0)
    hb = jnp.maximum(pb_ref[0][:, 64:] + pb_ref[1][:, 64:], 0.0)
    o_ref[...] = jnp.concatenate(
        [jnp.dot(ha, wa_ref[...], preferred_element_type=jnp.float32),
         jnp.dot(hb, wb_ref[...], preferred_element_type=jnp.float32)],
        axis=1)


def _enc2(pa, pb, wa, wb, block_rows, n_real):
    n_pad = pa.shape[1]
    return pl.pallas_call(
        _enc2_body,
        grid=(n_real // block_rows,),
        in_specs=[pl.BlockSpec((2, block_rows, 128), lambda i: (0, i, 0)),
                  pl.BlockSpec((2, block_rows, 128), lambda i: (0, i, 0)),
                  pl.BlockSpec((64, 64), lambda i: (0, 0)),
                  pl.BlockSpec((64, 64), lambda i: (0, 0))],
        out_specs=pl.BlockSpec((block_rows, 128), lambda i: (i, 0)),
        out_shape=jax.ShapeDtypeStruct((n_pad, 128), jnp.float32),
    )(pa, pb, wa, wb)


def _fin_body(ma_ref, mb_ref, n1_ref, n2_ref, wd_ref, bd_ref,
              z1_ref, z2_ref, z3_ref):
    ma = ma_ref[0][:, :64] + ma_ref[1][:, :64]
    mb = mb_ref[0][:, 64:] + mb_ref[1][:, 64:]
    zm1, zls1 = ma[:, :32], ma[:, 32:]
    zm2, zls2 = mb[:, :32], mb[:, 32:]
    z1_ref[...] = zm1 + n1_ref[...] * jnp.exp(zls1)
    z2_ref[...] = zm2 + n2_ref[...] * jnp.exp(zls2)
    z3_ref[...] = jnp.dot(zm1 + zm2, wd_ref[...],
                          preferred_element_type=jnp.float32) + bd_ref[...]


def _finalize(ma, mb, noise1, noise2, wd, bd, block_rows):
    n = noise1.shape[0]
    h2 = noise1.shape[1]
    sds = jax.ShapeDtypeStruct((n, h2), jnp.float32)
    return pl.pallas_call(
        _fin_body,
        grid=(n // block_rows,),
        in_specs=[pl.BlockSpec((2, block_rows, 128), lambda i: (0, i, 0)),
                  pl.BlockSpec((2, block_rows, 128), lambda i: (0, i, 0)),
                  pl.BlockSpec((block_rows, h2), lambda i: (i, 0)),
                  pl.BlockSpec((block_rows, h2), lambda i: (i, 0)),
                  pl.BlockSpec((h2, h2), lambda i: (0, 0)),
                  pl.BlockSpec((1, h2), lambda i: (0, 0))],
        out_specs=[pl.BlockSpec((block_rows, h2), lambda i: (i, 0)),
                   pl.BlockSpec((block_rows, h2), lambda i: (i, 0)),
                   pl.BlockSpec((block_rows, h2), lambda i: (i, 0))],
        out_shape=[sds, sds, sds],
    )(ma, mb, noise1, noise2, wd, bd.reshape(1, h2))


def _dec_body(l_ref, r_ref, o_ref):
    o_ref[...] = lax.dot_general(
        l_ref[...], r_ref[...], (((1,), (1,)), ((), ())),
        preferred_element_type=jnp.float32)


def _decode(z, block_rows):
    n, h2 = z.shape
    return pl.pallas_call(
        _dec_body,
        grid=(n // block_rows,),
        in_specs=[pl.BlockSpec((block_rows, h2), lambda i: (i, 0)),
                  pl.BlockSpec((n, h2), lambda i: (0, 0))],
        out_specs=pl.BlockSpec((block_rows, n), lambda i: (i, 0)),
        out_shape=jax.ShapeDtypeStruct((n, n), jnp.float32),
    )(z, z)


def kernel(features, graph1_edge_index, graph2_edge_index, noise1, noise2,
           W1_a, W2_a, W3_a, W1_b, W2_b, W3_b, Wd, bd):
    n, d = features.shape
    n_pad = -(-n // 128) * 128

    idx1, cpt1 = _prep_edges(graph1_edge_index, n)
    idx2, cpt2 = _prep_edges(graph2_edge_index, n)
    zeros_acc = jnp.zeros((n_pad, 128), jnp.float32)
    seg1 = _make_seg_sum(n_pad, 128, cpt1)
    seg2 = _make_seg_sum(n_pad, 128, cpt2)

    s_all = _matmul(features, jnp.concatenate([W1_a, W1_b], axis=1),
                    1000, n_pad)

    pa = seg1(s_all, idx1, zeros_acc)
    pb = seg2(s_all, idx2, zeros_acc)

    s23 = _enc2(pa, pb,
                jnp.concatenate([W2_a, W3_a], axis=1),
                jnp.concatenate([W2_b, W3_b], axis=1), 2000, n)

    ma = seg1(s23, idx1, zeros_acc)
    mb = seg2(s23, idx2, zeros_acc)

    z1, z2, z3 = _finalize(ma, mb, noise1, noise2, Wd, bd, 2000)

    rec1 = _decode(z1, 400).reshape(-1)
    rec2 = _decode(z2, 400).reshape(-1)
    return rec1, rec2, z3


# speedup vs baseline: 1.3977x; 1.3977x over previous
import functools

import jax
import jax.numpy as jnp
from jax import lax
from jax.experimental import pallas as pl
from jax.experimental.pallas import tpu as pltpu
from jax.experimental.pallas import tpu_sc as plsc

_NC = 2
_NS = 16
_NW = _NC * _NS
_CH = 128


@functools.lru_cache(maxsize=None)
def _make_seg_sum(n_pad: int, n_cols: int, edges_per_tile: int):
    rows_per_tile = n_pad // _NS
    n_chunks = edges_per_tile // _CH

    mesh = plsc.VectorSubcoreMesh(core_axis_name="c", subcore_axis_name="s")

    @functools.partial(
        pl.kernel,
        out_type=jax.ShapeDtypeStruct((2, n_pad, n_cols), jnp.float32),
        mesh=mesh,
        compiler_params=pltpu.CompilerParams(use_tc_tiling_on_sc=False),
        scratch_types=[
            pltpu.VMEM((_CH,), jnp.int32),
            pltpu.VMEM((_CH,), jnp.int32),
            pltpu.VMEM((_CH, n_cols), jnp.float32),
            pltpu.VMEM_SHARED((n_pad, n_cols), jnp.float32),
            pltpu.SemaphoreType.DMA,
        ],
    )
    def seg(table_hbm, src_hbm, dst_hbm, zeros_hbm, out_hbm,
            idx_s, idx_d, rows, acc_sp, sem):
        c = lax.axis_index("c")
        s = lax.axis_index("s")
        wid = s * _NC + c

        t0 = s * rows_per_tile
        pltpu.sync_copy(zeros_hbm.at[pl.ds(t0, rows_per_tile)],
                        acc_sp.at[pl.ds(t0, rows_per_tile)])
        plsc.subcore_barrier()

        base = wid * edges_per_tile

        def chunk(j, carry):
            off = pl.multiple_of(base + j * _CH, _CH)
            pltpu.sync_copy(src_hbm.at[pl.ds(off, _CH)], idx_s)
            pltpu.sync_copy(dst_hbm.at[pl.ds(off, _CH)], idx_d)
            pltpu.async_copy(table_hbm.at[idx_s], rows, sem).wait()
            pltpu.sync_copy(rows, acc_sp.at[idx_d], add=True)
            return carry

        lax.fori_loop(0, n_chunks, chunk, 0)
        plsc.subcore_barrier()

        pltpu.sync_copy(acc_sp.at[pl.ds(t0, rows_per_tile)],
                        out_hbm.at[c, pl.ds(t0, rows_per_tile)])

    return seg


def _prep_edges(edge_index, junk_row):
    e = edge_index.shape[1]
    edges_per_tile = -(-e // (_NW * _CH)) * _CH
    e_pad = _NW * edges_per_tile
    src = jnp.concatenate(
        [edge_index[0], jnp.zeros((e_pad - e,), jnp.int32)])
    dst = jnp.concatenate(
        [edge_index[1], jnp.full((e_pad - e,), junk_row, jnp.int32)])
    return src, dst, edges_per_tile


def _mm_body(x_ref, w_ref, o_ref):
    o_ref[...] = jnp.dot(x_ref[...], w_ref[...],
                         preferred_element_type=jnp.float32)


def _matmul(x, w, block_rows, n_out):
    n, d = x.shape
    k = w.shape[1]
    return pl.pallas_call(
        _mm_body,
        grid=(n // block_rows,),
        in_specs=[pl.BlockSpec((block_rows, d), lambda i: (i, 0)),
                  pl.BlockSpec((d, k), lambda i: (0, 0))],
        out_specs=pl.BlockSpec((block_rows, k), lambda i: (i, 0)),
        out_shape=jax.ShapeDtypeStruct((n_out, k), jnp.float32),
    )(x, w)


def _enc2_body(p_ref, w_ref, o_ref):
    h = jnp.maximum(p_ref[0] + p_ref[1], 0.0)
    o_ref[...] = jnp.dot(h, w_ref[...], preferred_element_type=jnp.float32)


def _enc2(parts, w23, block_rows, n_real):
    n_pad = parts.shape[1]
    k = w23.shape[1]
    return pl.pallas_call(
        _enc2_body,
        grid=(n_real // block_rows,),
        in_specs=[pl.BlockSpec((2, block_rows, 64), lambda i: (0, i, 0)),
                  pl.BlockSpec((64, k), lambda i: (0, 0))],
        out_specs=pl.BlockSpec((block_rows, k), lambda i: (i, 0)),
        out_shape=jax.ShapeDtypeStruct((n_pad, k), jnp.float32),
    )(parts, w23)


def _fin_body(ma_ref, mb_ref, n1_ref, n2_ref, wd_ref, bd_ref,
              z1_ref, z2_ref, z3_ref):
    ma = ma_ref[0] + ma_ref[1]
    mb = mb_ref[0] + mb_ref[1]
    zm1, zls1 = ma[:, :32], ma[:, 32:]
    zm2, zls2 = mb[:, :32], mb[:, 32:]
    z1_ref[...] = zm1 + n1_ref[...] * jnp.exp(zls1)
    z2_ref[...] = zm2 + n2_ref[...] * jnp.exp(zls2)
    z3_ref[...] = jnp.dot(zm1 + zm2, wd_ref[...],
                          preferred_element_type=jnp.float32) + bd_ref[...]


def _finalize(ma, mb, noise1, noise2, wd, bd, block_rows):
    n = noise1.shape[0]
    h2 = noise1.shape[1]
    sds = jax.ShapeDtypeStruct((n, h2), jnp.float32)
    return pl.pallas_call(
        _fin_body,
        grid=(n // block_rows,),
        in_specs=[pl.BlockSpec((2, block_rows, 64), lambda i: (0, i, 0)),
                  pl.BlockSpec((2, block_rows, 64), lambda i: (0, i, 0)),
                  pl.BlockSpec((block_rows, h2), lambda i: (i, 0)),
                  pl.BlockSpec((block_rows, h2), lambda i: (i, 0)),
                  pl.BlockSpec((h2, h2), lambda i: (0, 0)),
                  pl.BlockSpec((1, h2), lambda i: (0, 0))],
        out_specs=[pl.BlockSpec((block_rows, h2), lambda i: (i, 0)),
                   pl.BlockSpec((block_rows, h2), lambda i: (i, 0)),
                   pl.BlockSpec((block_rows, h2), lambda i: (i, 0))],
        out_shape=[sds, sds, sds],
    )(ma, mb, noise1, noise2, wd, bd.reshape(1, h2))


def _dec_body(l_ref, r_ref, o_ref):
    o_ref[...] = lax.dot_general(
        l_ref[...], r_ref[...], (((1,), (1,)), ((), ())),
        preferred_element_type=jnp.float32)


def _decode(z, block_rows):
    n, h2 = z.shape
    return pl.pallas_call(
        _dec_body,
        grid=(n // block_rows,),
        in_specs=[pl.BlockSpec((block_rows, h2), lambda i: (i, 0)),
                  pl.BlockSpec((n, h2), lambda i: (0, 0))],
        out_specs=pl.BlockSpec((block_rows, n), lambda i: (i, 0)),
        out_shape=jax.ShapeDtypeStruct((n, n), jnp.float32),
    )(z, z)


def kernel(features, graph1_edge_index, graph2_edge_index, noise1, noise2,
           W1_a, W2_a, W3_a, W1_b, W2_b, W3_b, Wd, bd):
    n, d = features.shape
    n_pad = -(-n // 128) * 128

    src1, dst1, cpt1 = _prep_edges(graph1_edge_index, n)
    src2, dst2, cpt2 = _prep_edges(graph2_edge_index, n)
    zeros_acc = jnp.zeros((n_pad, 64), jnp.float32)
    seg1 = _make_seg_sum(n_pad, 64, cpt1)
    seg2 = _make_seg_sum(n_pad, 64, cpt2)

    s_all = _matmul(features, jnp.concatenate([W1_a, W1_b], axis=1),
                    1000, n_pad)

    pa = seg1(s_all[:, :64], src1, dst1, zeros_acc)
    s23a = _enc2(pa, jnp.concatenate([W2_a, W3_a], axis=1), 2000, n)
    ma = seg1(s23a, src1, dst1, zeros_acc)

    pb = seg2(s_all[:, 64:], src2, dst2, zeros_acc)
    s23b = _enc2(pb, jnp.concatenate([W2_b, W3_b], axis=1), 2000, n)
    mb = seg2(s23b, src2, dst2, zeros_acc)

    z1, z2, z3 = _finalize(ma, mb, noise1, noise2, Wd, bd, 2000)

    rec1 = _decode(z1, 400).reshape(-1)
    rec2 = _decode(z2, 400).reshape(-1)
    return rec1, rec2, z3


# speedup vs baseline: 1.7318x; 1.2390x over previous
import functools

import jax
import jax.numpy as jnp
from jax import lax
from jax.experimental import pallas as pl
from jax.experimental.pallas import tpu as pltpu
from jax.experimental.pallas import tpu_sc as plsc

_NC = 2
_NS = 16
_NW = _NC * _NS
_CH = 128


@functools.lru_cache(maxsize=None)
def _make_seg_sum(n_pad: int, n_cols: int, edges_per_tile: int):
    rows_per_tile = n_pad // _NS
    n_chunks = edges_per_tile // _CH

    mesh = plsc.VectorSubcoreMesh(core_axis_name="c", subcore_axis_name="s")

    @functools.partial(
        pl.kernel,
        out_type=jax.ShapeDtypeStruct((2, n_pad, n_cols), jnp.float32),
        mesh=mesh,
        compiler_params=pltpu.CompilerParams(use_tc_tiling_on_sc=False),
        scratch_types=[
            pltpu.VMEM((_CH,), jnp.int32),
            pltpu.VMEM((_CH,), jnp.int32),
            pltpu.VMEM((_CH, n_cols), jnp.float32),
            pltpu.VMEM_SHARED((n_pad, n_cols), jnp.float32),
            pltpu.VMEM_SHARED((n_pad, n_cols), jnp.float32),
            pltpu.SemaphoreType.DMA,
        ],
    )
    def seg(table_hbm, src_hbm, dst_hbm, zeros_hbm, out_hbm,
            idx_s, idx_d, rows, acc_sp, table_sp, sem):
        c = lax.axis_index("c")
        s = lax.axis_index("s")
        wid = s * _NC + c

        t0 = s * rows_per_tile
        pltpu.sync_copy(zeros_hbm.at[pl.ds(t0, rows_per_tile)],
                        acc_sp.at[pl.ds(t0, rows_per_tile)])
        pltpu.sync_copy(table_hbm.at[pl.ds(t0, rows_per_tile)],
                        table_sp.at[pl.ds(t0, rows_per_tile)])
        plsc.subcore_barrier()

        base = wid * edges_per_tile

        def chunk(j, carry):
            off = pl.multiple_of(base + j * _CH, _CH)
            pltpu.sync_copy(src_hbm.at[pl.ds(off, _CH)], idx_s)
            pltpu.sync_copy(dst_hbm.at[pl.ds(off, _CH)], idx_d)
            pltpu.async_copy(table_sp.at[idx_s], rows, sem).wait()
            pltpu.sync_copy(rows, acc_sp.at[idx_d], add=True)
            return carry

        lax.fori_loop(0, n_chunks, chunk, 0)
        plsc.subcore_barrier()

        pltpu.sync_copy(acc_sp.at[pl.ds(t0, rows_per_tile)],
                        out_hbm.at[c, pl.ds(t0, rows_per_tile)])

    return seg


def _prep_edges(edge_index, junk_row):
    e = edge_index.shape[1]
    edges_per_tile = -(-e // (_NW * _CH)) * _CH
    e_pad = _NW * edges_per_tile
    src = jnp.concatenate(
        [edge_index[0], jnp.zeros((e_pad - e,), jnp.int32)])
    dst = jnp.concatenate(
        [edge_index[1], jnp.full((e_pad - e,), junk_row, jnp.int32)])
    return src, dst, edges_per_tile


def _mm_body(x_ref, w_ref, o_ref):
    o_ref[...] = jnp.dot(x_ref[...], w_ref[...],
                         preferred_element_type=jnp.float32)


def _matmul(x, w, block_rows, n_out):
    n, d = x.shape
    k = w.shape[1]
    return pl.pallas_call(
        _mm_body,
        grid=(n // block_rows,),
        in_specs=[pl.BlockSpec((block_rows, d), lambda i: (i, 0)),
                  pl.BlockSpec((d, k), lambda i: (0, 0))],
        out_specs=pl.BlockSpec((block_rows, k), lambda i: (i, 0)),
        out_shape=jax.ShapeDtypeStruct((n_out, k), jnp.float32),
    )(x, w)


def _enc2_body(p_ref, w_ref, o_ref):
    h = jnp.maximum(p_ref[0] + p_ref[1], 0.0)
    o_ref[...] = jnp.dot(h, w_ref[...], preferred_element_type=jnp.float32)


def _enc2(parts, w23, block_rows, n_real):
    n_pad = parts.shape[1]
    k = w23.shape[1]
    return pl.pallas_call(
        _enc2_body,
        grid=(n_real // block_rows,),
        in_specs=[pl.BlockSpec((2, block_rows, 64), lambda i: (0, i, 0)),
                  pl.BlockSpec((64, k), lambda i: (0, 0))],
        out_specs=pl.BlockSpec((block_rows, k), lambda i: (i, 0)),
        out_shape=jax.ShapeDtypeStruct((n_pad, k), jnp.float32),
    )(parts, w23)


def _fin_body(ma_ref, mb_ref, n1_ref, n2_ref, wd_ref, bd_ref,
              z1_ref, z2_ref, z3_ref):
    ma = ma_ref[0] + ma_ref[1]
    mb = mb_ref[0] + mb_ref[1]
    zm1, zls1 = ma[:, :32], ma[:, 32:]
    zm2, zls2 = mb[:, :32], mb[:, 32:]
    z1_ref[...] = zm1 + n1_ref[...] * jnp.exp(zls1)
    z2_ref[...] = zm2 + n2_ref[...] * jnp.exp(zls2)
    z3_ref[...] = jnp.dot(zm1 + zm2, wd_ref[...],
                          preferred_element_type=jnp.float32) + bd_ref[...]


def _finalize(ma, mb, noise1, noise2, wd, bd, block_rows):
    n = noise1.shape[0]
    h2 = noise1.shape[1]
    sds = jax.ShapeDtypeStruct((n, h2), jnp.float32)
    return pl.pallas_call(
        _fin_body,
        grid=(n // block_rows,),
        in_specs=[pl.BlockSpec((2, block_rows, 64), lambda i: (0, i, 0)),
                  pl.BlockSpec((2, block_rows, 64), lambda i: (0, i, 0)),
                  pl.BlockSpec((block_rows, h2), lambda i: (i, 0)),
                  pl.BlockSpec((block_rows, h2), lambda i: (i, 0)),
                  pl.BlockSpec((h2, h2), lambda i: (0, 0)),
                  pl.BlockSpec((1, h2), lambda i: (0, 0))],
        out_specs=[pl.BlockSpec((block_rows, h2), lambda i: (i, 0)),
                   pl.BlockSpec((block_rows, h2), lambda i: (i, 0)),
                   pl.BlockSpec((block_rows, h2), lambda i: (i, 0))],
        out_shape=[sds, sds, sds],
    )(ma, mb, noise1, noise2, wd, bd.reshape(1, h2))


def _dec_body(l_ref, r_ref, o_ref):
    o_ref[...] = lax.dot_general(
        l_ref[...], r_ref[...], (((1,), (1,)), ((), ())),
        preferred_element_type=jnp.float32)


def _decode(z, block_rows):
    n, h2 = z.shape
    return pl.pallas_call(
        _dec_body,
        grid=(n // block_rows,),
        in_specs=[pl.BlockSpec((block_rows, h2), lambda i: (i, 0)),
                  pl.BlockSpec((n, h2), lambda i: (0, 0))],
        out_specs=pl.BlockSpec((block_rows, n), lambda i: (i, 0)),
        out_shape=jax.ShapeDtypeStruct((n, n), jnp.float32),
    )(z, z)


def kernel(features, graph1_edge_index, graph2_edge_index, noise1, noise2,
           W1_a, W2_a, W3_a, W1_b, W2_b, W3_b, Wd, bd):
    n, d = features.shape
    n_pad = -(-n // 128) * 128

    src1, dst1, cpt1 = _prep_edges(graph1_edge_index, n)
    src2, dst2, cpt2 = _prep_edges(graph2_edge_index, n)
    zeros_acc = jnp.zeros((n_pad, 64), jnp.float32)
    seg1 = _make_seg_sum(n_pad, 64, cpt1)
    seg2 = _make_seg_sum(n_pad, 64, cpt2)

    s_all = _matmul(features, jnp.concatenate([W1_a, W1_b], axis=1),
                    1000, n_pad)

    pa = seg1(s_all[:, :64], src1, dst1, zeros_acc)
    s23a = _enc2(pa, jnp.concatenate([W2_a, W3_a], axis=1), 2000, n)
    ma = seg1(s23a, src1, dst1, zeros_acc)

    pb = seg2(s_all[:, 64:], src2, dst2, zeros_acc)
    s23b = _enc2(pb, jnp.concatenate([W2_b, W3_b], axis=1), 2000, n)
    mb = seg2(s23b, src2, dst2, zeros_acc)

    z1, z2, z3 = _finalize(ma, mb, noise1, noise2, Wd, bd, 2000)

    rec1 = _decode(z1, 400).reshape(-1)
    rec2 = _decode(z2, 400).reshape(-1)
    return rec1, rec2, z3


# speedup vs baseline: 1.8675x; 1.0784x over previous
import functools

import jax
import jax.numpy as jnp
from jax import lax
from jax.experimental import pallas as pl
from jax.experimental.pallas import tpu as pltpu
from jax.experimental.pallas import tpu_sc as plsc

_NC = 2
_NS = 16
_NW = _NC * _NS
_CH = 128


@functools.lru_cache(maxsize=None)
def _make_seg_sum(n_pad: int, n_cols: int, edges_per_tile: int):
    rows_per_tile = n_pad // _NS
    n_chunks = edges_per_tile // _CH

    mesh = plsc.VectorSubcoreMesh(core_axis_name="c", subcore_axis_name="s")

    @functools.partial(
        pl.kernel,
        out_type=jax.ShapeDtypeStruct((2, n_pad, n_cols), jnp.float32),
        mesh=mesh,
        compiler_params=pltpu.CompilerParams(use_tc_tiling_on_sc=False),
        scratch_types=[
            pltpu.VMEM((2 * _CH,), jnp.int32),
            pltpu.VMEM((_CH, n_cols), jnp.float32),
            pltpu.VMEM_SHARED((n_pad, n_cols), jnp.float32),
            pltpu.VMEM_SHARED((n_pad, n_cols), jnp.float32),
            pltpu.SemaphoreType.DMA,
        ],
    )
    def seg(table_hbm, idx_hbm, zeros_hbm, out_hbm,
            idx, rows, acc_sp, table_sp, sem):
        c = lax.axis_index("c")
        s = lax.axis_index("s")
        wid = s * _NC + c

        t0 = s * rows_per_tile
        pltpu.sync_copy(zeros_hbm.at[pl.ds(t0, rows_per_tile)],
                        acc_sp.at[pl.ds(t0, rows_per_tile)])
        pltpu.sync_copy(table_hbm.at[pl.ds(t0, rows_per_tile)],
                        table_sp.at[pl.ds(t0, rows_per_tile)])
        plsc.subcore_barrier()

        base2 = wid * 2 * edges_per_tile

        def chunk(j, carry):
            off = pl.multiple_of(base2 + j * 2 * _CH, 2 * _CH)
            pltpu.sync_copy(idx_hbm.at[pl.ds(off, 2 * _CH)], idx)
            pltpu.async_copy(table_sp.at[idx.at[pl.ds(0, _CH)]],
                             rows, sem).wait()
            pltpu.sync_copy(rows, acc_sp.at[idx.at[pl.ds(_CH, _CH)]],
                            add=True)
            return carry

        lax.fori_loop(0, n_chunks, chunk, 0)
        plsc.subcore_barrier()

        pltpu.sync_copy(acc_sp.at[pl.ds(t0, rows_per_tile)],
                        out_hbm.at[c, pl.ds(t0, rows_per_tile)])

    return seg


def _prep_edges(edge_index, junk_row):
    e = edge_index.shape[1]
    edges_per_tile = -(-e // (_NW * _CH)) * _CH
    e_pad = _NW * edges_per_tile
    src = jnp.concatenate(
        [edge_index[0], jnp.zeros((e_pad - e,), jnp.int32)])
    dst = jnp.concatenate(
        [edge_index[1], jnp.full((e_pad - e,), junk_row, jnp.int32)])
    inter = jnp.stack(
        [src.reshape(-1, _CH), dst.reshape(-1, _CH)], axis=1).reshape(-1)
    return inter, edges_per_tile


def _mm_body(x_ref, w_ref, o_ref):
    o_ref[...] = jnp.dot(x_ref[...], w_ref[...],
                         preferred_element_type=jnp.float32)


def _matmul(x, w, block_rows, n_out):
    n, d = x.shape
    k = w.shape[1]
    return pl.pallas_call(
        _mm_body,
        grid=(n // block_rows,),
        in_specs=[pl.BlockSpec((block_rows, d), lambda i: (i, 0)),
                  pl.BlockSpec((d, k), lambda i: (0, 0))],
        out_specs=pl.BlockSpec((block_rows, k), lambda i: (i, 0)),
        out_shape=jax.ShapeDtypeStruct((n_out, k), jnp.float32),
    )(x, w)


def _enc2_body(p_ref, w_ref, o_ref):
    h = jnp.maximum(p_ref[0] + p_ref[1], 0.0)
    o_ref[...] = jnp.dot(h, w_ref[...], preferred_element_type=jnp.float32)


def _enc2(parts, w23, block_rows, n_real):
    n_pad = parts.shape[1]
    k = w23.shape[1]
    return pl.pallas_call(
        _enc2_body,
        grid=(n_real // block_rows,),
        in_specs=[pl.BlockSpec((2, block_rows, 64), lambda i: (0, i, 0)),
                  pl.BlockSpec((64, k), lambda i: (0, 0))],
        out_specs=pl.BlockSpec((block_rows, k), lambda i: (i, 0)),
        out_shape=jax.ShapeDtypeStruct((n_pad, k), jnp.float32),
    )(parts, w23)


def _fin_body(ma_ref, mb_ref, n1_ref, n2_ref, wd_ref, bd_ref,
              z1_ref, z2_ref, z3_ref):
    ma = ma_ref[0] + ma_ref[1]
    mb = mb_ref[0] + mb_ref[1]
    zm1, zls1 = ma[:, :32], ma[:, 32:]
    zm2, zls2 = mb[:, :32], mb[:, 32:]
    z1_ref[...] = zm1 + n1_ref[...] * jnp.exp(zls1)
    z2_ref[...] = zm2 + n2_ref[...] * jnp.exp(zls2)
    z3_ref[...] = jnp.dot(zm1 + zm2, wd_ref[...],
                          preferred_element_type=jnp.float32) + bd_ref[...]


def _finalize(ma, mb, noise1, noise2, wd, bd, block_rows):
    n = noise1.shape[0]
    h2 = noise1.shape[1]
    sds = jax.ShapeDtypeStruct((n, h2), jnp.float32)
    return pl.pallas_call(
        _fin_body,
        grid=(n // block_rows,),
        in_specs=[pl.BlockSpec((2, block_rows, 64), lambda i: (0, i, 0)),
                  pl.BlockSpec((2, block_rows, 64), lambda i: (0, i, 0)),
                  pl.BlockSpec((block_rows, h2), lambda i: (i, 0)),
                  pl.BlockSpec((block_rows, h2), lambda i: (i, 0)),
                  pl.BlockSpec((h2, h2), lambda i: (0, 0)),
                  pl.BlockSpec((1, h2), lambda i: (0, 0))],
        out_specs=[pl.BlockSpec((block_rows, h2), lambda i: (i, 0)),
                   pl.BlockSpec((block_rows, h2), lambda i: (i, 0)),
                   pl.BlockSpec((block_rows, h2), lambda i: (i, 0))],
        out_shape=[sds, sds, sds],
    )(ma, mb, noise1, noise2, wd, bd.reshape(1, h2))


def _dec_body(l_ref, r_ref, o_ref):
    o_ref[...] = lax.dot_general(
        l_ref[...], r_ref[...], (((1,), (1,)), ((), ())),
        preferred_element_type=jnp.float32)


def _decode(z, block_rows):
    n, h2 = z.shape
    return pl.pallas_call(
        _dec_body,
        grid=(n // block_rows,),
        in_specs=[pl.BlockSpec((block_rows, h2), lambda i: (i, 0)),
                  pl.BlockSpec((n, h2), lambda i: (0, 0))],
        out_specs=pl.BlockSpec((block_rows, n), lambda i: (i, 0)),
        out_shape=jax.ShapeDtypeStruct((n, n), jnp.float32),
    )(z, z)


def kernel(features, graph1_edge_index, graph2_edge_index, noise1, noise2,
           W1_a, W2_a, W3_a, W1_b, W2_b, W3_b, Wd, bd):
    n, d = features.shape
    n_pad = -(-n // 128) * 128

    idx1, cpt1 = _prep_edges(graph1_edge_index, n)
    idx2, cpt2 = _prep_edges(graph2_edge_index, n)
    zeros_acc = jnp.zeros((n_pad, 64), jnp.float32)
    seg1 = _make_seg_sum(n_pad, 64, cpt1)
    seg2 = _make_seg_sum(n_pad, 64, cpt2)

    s_all = _matmul(features, jnp.concatenate([W1_a, W1_b], axis=1),
                    1000, n_pad)

    pa = seg1(s_all[:, :64], idx1, zeros_acc)
    s23a = _enc2(pa, jnp.concatenate([W2_a, W3_a], axis=1), 2000, n)
    ma = seg1(s23a, idx1, zeros_acc)

    pb = seg2(s_all[:, 64:], idx2, zeros_acc)
    s23b = _enc2(pb, jnp.concatenate([W2_b, W3_b], axis=1), 2000, n)
    mb = seg2(s23b, idx2, zeros_acc)

    z1, z2, z3 = _finalize(ma, mb, noise1, noise2, Wd, bd, 2000)

    rec1 = _decode(z1, 400).reshape(-1)
    rec2 = _decode(z2, 400).reshape(-1)
    return rec1, rec2, z3


# speedup vs baseline: 1.9524x; 1.0455x over previous
import functools

import jax
import jax.numpy as jnp
from jax import lax
from jax.experimental import pallas as pl
from jax.experimental.pallas import tpu as pltpu
from jax.experimental.pallas import tpu_sc as plsc

_NC = 2
_NS = 16
_NW = _NC * _NS
_CH = 128


@functools.lru_cache(maxsize=None)
def _make_seg_sum(n_pad: int, n_cols: int, edges_per_tile: int):
    rows_per_tile = n_pad // _NS
    n_chunks = edges_per_tile // _CH

    mesh = plsc.VectorSubcoreMesh(core_axis_name="c", subcore_axis_name="s")

    @functools.partial(
        pl.kernel,
        out_type=jax.ShapeDtypeStruct((2, n_pad, n_cols), jnp.float32),
        mesh=mesh,
        compiler_params=pltpu.CompilerParams(use_tc_tiling_on_sc=False),
        scratch_types=[
            pltpu.VMEM((2 * edges_per_tile,), jnp.int32),
            pltpu.VMEM((_CH, n_cols), jnp.float32),
            pltpu.VMEM_SHARED((n_pad, n_cols), jnp.float32),
            pltpu.VMEM_SHARED((n_pad, n_cols), jnp.float32),
            pltpu.SemaphoreType.DMA,
        ],
    )
    def seg(table_hbm, idx_hbm, zeros_hbm, out_hbm,
            idx, rows, acc_sp, table_sp, sem):
        c = lax.axis_index("c")
        s = lax.axis_index("s")
        wid = s * _NC + c

        t0 = s * rows_per_tile
        pltpu.sync_copy(zeros_hbm.at[pl.ds(t0, rows_per_tile)],
                        acc_sp.at[pl.ds(t0, rows_per_tile)])
        pltpu.sync_copy(table_hbm.at[pl.ds(t0, rows_per_tile)],
                        table_sp.at[pl.ds(t0, rows_per_tile)])
        plsc.subcore_barrier()

        base2 = wid * 2 * edges_per_tile
        pltpu.sync_copy(idx_hbm.at[pl.ds(pl.multiple_of(base2, 2 * _CH),
                                         2 * edges_per_tile)], idx)

        def chunk(j, carry):
            off = pl.multiple_of(j * 2 * _CH, 2 * _CH)
            pltpu.async_copy(table_sp.at[idx.at[pl.ds(off, _CH)]],
                             rows, sem).wait()
            pltpu.sync_copy(rows, acc_sp.at[idx.at[pl.ds(off + _CH, _CH)]],
                            add=True)
            return carry

        lax.fori_loop(0, n_chunks, chunk, 0)
        plsc.subcore_barrier()

        pltpu.sync_copy(acc_sp.at[pl.ds(t0, rows_per_tile)],
                        out_hbm.at[c, pl.ds(t0, rows_per_tile)])

    return seg


def _prep_edges(edge_index, junk_row):
    e = edge_index.shape[1]
    edges_per_tile = -(-e // (_NW * _CH)) * _CH
    e_pad = _NW * edges_per_tile
    src = jnp.concatenate(
        [edge_index[0], jnp.zeros((e_pad - e,), jnp.int32)])
    dst = jnp.concatenate(
        [edge_index[1], jnp.full((e_pad - e,), junk_row, jnp.int32)])
    inter = jnp.stack(
        [src.reshape(-1, _CH), dst.reshape(-1, _CH)], axis=1).reshape(-1)
    return inter, edges_per_tile


def _mm_body(x_ref, w_ref, o_ref):
    o_ref[...] = jnp.dot(x_ref[...], w_ref[...],
                         preferred_element_type=jnp.float32)


def _matmul(x, w, block_rows, n_out):
    n, d = x.shape
    k = w.shape[1]
    return pl.pallas_call(
        _mm_body,
        grid=(n // block_rows,),
        in_specs=[pl.BlockSpec((block_rows, d), lambda i: (i, 0)),
                  pl.BlockSpec((d, k), lambda i: (0, 0))],
        out_specs=pl.BlockSpec((block_rows, k), lambda i: (i, 0)),
        out_shape=jax.ShapeDtypeStruct((n_out, k), jnp.float32),
    )(x, w)


def _enc2_body(p_ref, w_ref, o_ref):
    h = jnp.maximum(p_ref[0] + p_ref[1], 0.0)
    o_ref[...] = jnp.dot(h, w_ref[...], preferred_element_type=jnp.float32)


def _enc2(parts, w23, block_rows, n_real):
    n_pad = parts.shape[1]
    k = w23.shape[1]
    return pl.pallas_call(
        _enc2_body,
        grid=(n_real // block_rows,),
        in_specs=[pl.BlockSpec((2, block_rows, 64), lambda i: (0, i, 0)),
                  pl.BlockSpec((64, k), lambda i: (0, 0))],
        out_specs=pl.BlockSpec((block_rows, k), lambda i: (i, 0)),
        out_shape=jax.ShapeDtypeStruct((n_pad, k), jnp.float32),
    )(parts, w23)


def _fin_body(ma_ref, mb_ref, n1_ref, n2_ref, wd_ref, bd_ref,
              z1_ref, z2_ref, z3_ref):
    ma = ma_ref[0] + ma_ref[1]
    mb = mb_ref[0] + mb_ref[1]
    zm1, zls1 = ma[:, :32], ma[:, 32:]
    zm2, zls2 = mb[:, :32], mb[:, 32:]
    z1_ref[...] = zm1 + n1_ref[...] * jnp.exp(zls1)
    z2_ref[...] = zm2 + n2_ref[...] * jnp.exp(zls2)
    z3_ref[...] = jnp.dot(zm1 + zm2, wd_ref[...],
                          preferred_element_type=jnp.float32) + bd_ref[...]


def _finalize(ma, mb, noise1, noise2, wd, bd, block_rows):
    n = noise1.shape[0]
    h2 = noise1.shape[1]
    sds = jax.ShapeDtypeStruct((n, h2), jnp.float32)
    return pl.pallas_call(
        _fin_body,
        grid=(n // block_rows,),
        in_specs=[pl.BlockSpec((2, block_rows, 64), lambda i: (0, i, 0)),
                  pl.BlockSpec((2, block_rows, 64), lambda i: (0, i, 0)),
                  pl.BlockSpec((block_rows, h2), lambda i: (i, 0)),
                  pl.BlockSpec((block_rows, h2), lambda i: (i, 0)),
                  pl.BlockSpec((h2, h2), lambda i: (0, 0)),
                  pl.BlockSpec((1, h2), lambda i: (0, 0))],
        out_specs=[pl.BlockSpec((block_rows, h2), lambda i: (i, 0)),
                   pl.BlockSpec((block_rows, h2), lambda i: (i, 0)),
                   pl.BlockSpec((block_rows, h2), lambda i: (i, 0))],
        out_shape=[sds, sds, sds],
    )(ma, mb, noise1, noise2, wd, bd.reshape(1, h2))


def _dec_body(l_ref, r_ref, o_ref):
    o_ref[...] = lax.dot_general(
        l_ref[...], r_ref[...], (((1,), (1,)), ((), ())),
        preferred_element_type=jnp.float32)


def _decode(z, block_rows):
    n, h2 = z.shape
    return pl.pallas_call(
        _dec_body,
        grid=(n // block_rows,),
        in_specs=[pl.BlockSpec((block_rows, h2), lambda i: (i, 0)),
                  pl.BlockSpec((n, h2), lambda i: (0, 0))],
        out_specs=pl.BlockSpec((block_rows, n), lambda i: (i, 0)),
        out_shape=jax.ShapeDtypeStruct((n, n), jnp.float32),
    )(z, z)


def kernel(features, graph1_edge_index, graph2_edge_index, noise1, noise2,
           W1_a, W2_a, W3_a, W1_b, W2_b, W3_b, Wd, bd):
    n, d = features.shape
    n_pad = -(-n // 128) * 128

    idx1, cpt1 = _prep_edges(graph1_edge_index, n)
    idx2, cpt2 = _prep_edges(graph2_edge_index, n)
    zeros_acc = jnp.zeros((n_pad, 64), jnp.float32)
    seg1 = _make_seg_sum(n_pad, 64, cpt1)
    seg2 = _make_seg_sum(n_pad, 64, cpt2)

    s_all = _matmul(features, jnp.concatenate([W1_a, W1_b], axis=1),
                    1000, n_pad)

    pa = seg1(s_all[:, :64], idx1, zeros_acc)
    s23a = _enc2(pa, jnp.concatenate([W2_a, W3_a], axis=1), 2000, n)
    ma = seg1(s23a, idx1, zeros_acc)

    pb = seg2(s_all[:, 64:], idx2, zeros_acc)
    s23b = _enc2(pb, jnp.concatenate([W2_b, W3_b], axis=1), 2000, n)
    mb = seg2(s23b, idx2, zeros_acc)

    z1, z2, z3 = _finalize(ma, mb, noise1, noise2, Wd, bd, 2000)

    rec1 = _decode(z1, 400).reshape(-1)
    rec2 = _decode(z2, 400).reshape(-1)
    return rec1, rec2, z3


# speedup vs baseline: 1.9638x; 1.0058x over previous
import functools

import jax
import jax.numpy as jnp
from jax import lax
from jax.experimental import pallas as pl
from jax.experimental.pallas import tpu as pltpu
from jax.experimental.pallas import tpu_sc as plsc

_NC = 2
_NS = 16
_NW = _NC * _NS
_CH = 128


@functools.lru_cache(maxsize=None)
def _make_seg_sum(n_pad: int, n_cols: int, edges_per_tile: int):
    rows_per_tile = n_pad // _NS
    n_chunks = edges_per_tile // _CH

    mesh = plsc.VectorSubcoreMesh(core_axis_name="c", subcore_axis_name="s")

    @functools.partial(
        pl.kernel,
        out_type=jax.ShapeDtypeStruct((2, n_pad, n_cols), jnp.float32),
        mesh=mesh,
        compiler_params=pltpu.CompilerParams(use_tc_tiling_on_sc=False),
        scratch_types=[
            pltpu.VMEM((2 * edges_per_tile,), jnp.int32),
            [pltpu.VMEM((_CH, n_cols), jnp.float32)
             for _ in range(2)],
            pltpu.VMEM_SHARED((n_pad, n_cols), jnp.float32),
            pltpu.VMEM_SHARED((n_pad, n_cols), jnp.float32),
            [pltpu.SemaphoreType.DMA for _ in range(2)],
        ],
    )
    def seg(table_hbm, idx_hbm, zeros_hbm, out_hbm,
            idx, rows, acc_sp, table_sp, sems):
        c = lax.axis_index("c")
        s = lax.axis_index("s")
        wid = s * _NC + c

        t0 = s * rows_per_tile
        pltpu.sync_copy(zeros_hbm.at[pl.ds(t0, rows_per_tile)],
                        acc_sp.at[pl.ds(t0, rows_per_tile)])
        pltpu.sync_copy(table_hbm.at[pl.ds(t0, rows_per_tile)],
                        table_sp.at[pl.ds(t0, rows_per_tile)])
        plsc.subcore_barrier()

        base2 = wid * 2 * edges_per_tile
        pltpu.sync_copy(idx_hbm.at[pl.ds(pl.multiple_of(base2, 2 * _CH),
                                         2 * edges_per_tile)], idx)

        sem0, sem1 = sems
        rows0, rows1 = rows

        def pair(p, carry):
            o0 = pl.multiple_of(p * 4 * _CH, 2 * _CH)
            o1 = o0 + 2 * _CH
            g0 = pltpu.async_copy(table_sp.at[idx.at[pl.ds(o0, _CH)]],
                                  rows0, sem0)
            g1 = pltpu.async_copy(table_sp.at[idx.at[pl.ds(o1, _CH)]],
                                  rows1, sem1)
            g0.wait()
            pltpu.sync_copy(rows0, acc_sp.at[idx.at[pl.ds(o0 + _CH, _CH)]],
                            add=True)
            g1.wait()
            pltpu.sync_copy(rows1, acc_sp.at[idx.at[pl.ds(o1 + _CH, _CH)]],
                            add=True)
            return carry

        lax.fori_loop(0, n_chunks // 2, pair, 0)
        plsc.subcore_barrier()

        pltpu.sync_copy(acc_sp.at[pl.ds(t0, rows_per_tile)],
                        out_hbm.at[c, pl.ds(t0, rows_per_tile)])

    return seg


def _prep_edges(edge_index, junk_row):
    e = edge_index.shape[1]
    edges_per_tile = -(-e // (_NW * _CH)) * _CH
    e_pad = _NW * edges_per_tile
    src = jnp.concatenate(
        [edge_index[0], jnp.zeros((e_pad - e,), jnp.int32)])
    dst = jnp.concatenate(
        [edge_index[1], jnp.full((e_pad - e,), junk_row, jnp.int32)])
    inter = jnp.stack(
        [src.reshape(-1, _CH), dst.reshape(-1, _CH)], axis=1).reshape(-1)
    return inter, edges_per_tile


def _mm_body(x_ref, w_ref, o_ref):
    o_ref[...] = jnp.dot(x_ref[...], w_ref[...],
                         preferred_element_type=jnp.float32)


def _matmul(x, w, block_rows, n_out):
    n, d = x.shape
    k = w.shape[1]
    return pl.pallas_call(
        _mm_body,
        grid=(n // block_rows,),
        in_specs=[pl.BlockSpec((block_rows, d), lambda i: (i, 0)),
                  pl.BlockSpec((d, k), lambda i: (0, 0))],
        out_specs=pl.BlockSpec((block_rows, k), lambda i: (i, 0)),
        out_shape=jax.ShapeDtypeStruct((n_out, k), jnp.float32),
    )(x, w)


def _enc2_body(p_ref, w_ref, o_ref):
    h = jnp.maximum(p_ref[0] + p_ref[1], 0.0)
    o_ref[...] = jnp.dot(h, w_ref[...], preferred_element_type=jnp.float32)


def _enc2(parts, w23, block_rows, n_real):
    n_pad = parts.shape[1]
    k = w23.shape[1]
    return pl.pallas_call(
        _enc2_body,
        grid=(n_real // block_rows,),
        in_specs=[pl.BlockSpec((2, block_rows, 64), lambda i: (0, i, 0)),
                  pl.BlockSpec((64, k), lambda i: (0, 0))],
        out_specs=pl.BlockSpec((block_rows, k), lambda i: (i, 0)),
        out_shape=jax.ShapeDtypeStruct((n_pad, k), jnp.float32),
    )(parts, w23)


def _fin_body(ma_ref, mb_ref, n1_ref, n2_ref, wd_ref, bd_ref,
              z1_ref, z2_ref, z3_ref):
    ma = ma_ref[0] + ma_ref[1]
    mb = mb_ref[0] + mb_ref[1]
    zm1, zls1 = ma[:, :32], ma[:, 32:]
    zm2, zls2 = mb[:, :32], mb[:, 32:]
    z1_ref[...] = zm1 + n1_ref[...] * jnp.exp(zls1)
    z2_ref[...] = zm2 + n2_ref[...] * jnp.exp(zls2)
    z3_ref[...] = jnp.dot(zm1 + zm2, wd_ref[...],
                          preferred_element_type=jnp.float32) + bd_ref[...]


def _finalize(ma, mb, noise1, noise2, wd, bd, block_rows):
    n = noise1.shape[0]
    h2 = noise1.shape[1]
    sds = jax.ShapeDtypeStruct((n, h2), jnp.float32)
    return pl.pallas_call(
        _fin_body,
        grid=(n // block_rows,),
        in_specs=[pl.BlockSpec((2, block_rows, 64), lambda i: (0, i, 0)),
                  pl.BlockSpec((2, block_rows, 64), lambda i: (0, i, 0)),
                  pl.BlockSpec((block_rows, h2), lambda i: (i, 0)),
                  pl.BlockSpec((block_rows, h2), lambda i: (i, 0)),
                  pl.BlockSpec((h2, h2), lambda i: (0, 0)),
                  pl.BlockSpec((1, h2), lambda i: (0, 0))],
        out_specs=[pl.BlockSpec((block_rows, h2), lambda i: (i, 0)),
                   pl.BlockSpec((block_rows, h2), lambda i: (i, 0)),
                   pl.BlockSpec((block_rows, h2), lambda i: (i, 0))],
        out_shape=[sds, sds, sds],
    )(ma, mb, noise1, noise2, wd, bd.reshape(1, h2))


def _dec_body(l_ref, r_ref, o_ref):
    o_ref[...] = lax.dot_general(
        l_ref[...], r_ref[...], (((1,), (1,)), ((), ())),
        preferred_element_type=jnp.float32)


def _decode(z, block_rows):
    n, h2 = z.shape
    return pl.pallas_call(
        _dec_body,
        grid=(n // block_rows,),
        in_specs=[pl.BlockSpec((block_rows, h2), lambda i: (i, 0)),
                  pl.BlockSpec((n, h2), lambda i: (0, 0))],
        out_specs=pl.BlockSpec((block_rows, n), lambda i: (i, 0)),
        out_shape=jax.ShapeDtypeStruct((n, n), jnp.float32),
    )(z, z)


def kernel(features, graph1_edge_index, graph2_edge_index, noise1, noise2,
           W1_a, W2_a, W3_a, W1_b, W2_b, W3_b, Wd, bd):
    n, d = features.shape
    n_pad = -(-n // 128) * 128

    idx1, cpt1 = _prep_edges(graph1_edge_index, n)
    idx2, cpt2 = _prep_edges(graph2_edge_index, n)
    zeros_acc = jnp.zeros((n_pad, 64), jnp.float32)
    seg1 = _make_seg_sum(n_pad, 64, cpt1)
    seg2 = _make_seg_sum(n_pad, 64, cpt2)

    s_all = _matmul(features, jnp.concatenate([W1_a, W1_b], axis=1),
                    1000, n_pad)

    pa = seg1(s_all[:, :64], idx1, zeros_acc)
    s23a = _enc2(pa, jnp.concatenate([W2_a, W3_a], axis=1), 2000, n)
    ma = seg1(s23a, idx1, zeros_acc)

    pb = seg2(s_all[:, 64:], idx2, zeros_acc)
    s23b = _enc2(pb, jnp.concatenate([W2_b, W3_b], axis=1), 2000, n)
    mb = seg2(s23b, idx2, zeros_acc)

    z1, z2, z3 = _finalize(ma, mb, noise1, noise2, Wd, bd, 2000)

    rec1 = _decode(z1, 400).reshape(-1)
    rec2 = _decode(z2, 400).reshape(-1)
    return rec1, rec2, z3
